# Initial kernel scaffold; baseline (speedup 1.0000x reference)
import jax
import jax.numpy as jnp
from jax.experimental import pallas as pl


def kernel(x, edge_index, W1, b1, W2, b2, W3, b3):
    raise NotImplementedError("write your pallas kernel here")


# speedup vs baseline: 12.0952x; 12.0952x over previous
import functools

import jax
import jax.numpy as jnp
from jax import lax
from jax.experimental import pallas as pl
from jax.experimental.pallas import tpu as pltpu
from jax.experimental.pallas import tpu_sc as plsc

N = 10000
D = 128
E = 320000

NC = 2
NS = 16
NW = NC * NS
LANES = 16

NP = 10112
RPT = NP // NS
K = 128
E_PAD = 323584
EW = E_PAD // NW
CH = EW // K

_mesh = plsc.VectorSubcoreMesh(core_axis_name="c", subcore_axis_name="s")
_sc_params = pltpu.CompilerParams(needs_layout_passes=False)


@functools.partial(
    pl.kernel,
    out_type=jax.ShapeDtypeStruct((NW * NP,), jnp.float32),
    mesh=_mesh,
    scratch_types=[
        pltpu.VMEM((K,), jnp.int32),
        pltpu.VMEM((K,), jnp.int32),
        pltpu.VMEM((NP,), jnp.float32),
        pltpu.VMEM((NP,), jnp.float32),
    ],
    compiler_params=_sc_params,
)
def _sc_scalar(table_hbm, src_hbm, dst_hbm, zeros_hbm, out_hbm,
               src_v, dst_v, table_v, acc_v):
    c = lax.axis_index("c")
    s = lax.axis_index("s")
    wid = s * NC + c
    pltpu.sync_copy(zeros_hbm, acc_v)
    pltpu.sync_copy(table_hbm, table_v)
    base = wid * EW

    def chunk(j, _):
        off = base + j * K
        pltpu.sync_copy(src_hbm.at[pl.ds(off, K)], src_v)
        pltpu.sync_copy(dst_hbm.at[pl.ds(off, K)], dst_v)
        for i in range(K // LANES):
            sv = src_v[pl.ds(i * LANES, LANES)]
            dv = dst_v[pl.ds(i * LANES, LANES)]
            vals = plsc.load_gather(table_v, [sv])
            plsc.addupdate_scatter(acc_v, [dv], vals)
        return 0

    lax.fori_loop(0, CH, chunk, 0)
    pltpu.sync_copy(acc_v, out_hbm.at[pl.ds(wid * NP, NP)])


@functools.partial(
    pl.kernel,
    out_type=jax.ShapeDtypeStruct((NC, NP, D), jnp.float32),
    mesh=_mesh,
    scratch_types=[
        pltpu.VMEM((K,), jnp.int32),
        pltpu.VMEM((K,), jnp.int32),
        pltpu.VMEM((K, D), jnp.float32),
        pltpu.VMEM_SHARED((NP, D), jnp.float32),
        pltpu.SemaphoreType.DMA,
    ],
    compiler_params=_sc_params,
)
def _sc_rows(y_hbm, src_hbm, dst_hbm, zeros_hbm, out_hbm,
             src_v, dst_v, rows_v, acc_sh, sem):
    c = lax.axis_index("c")
    s = lax.axis_index("s")
    wid = s * NC + c
    pltpu.sync_copy(zeros_hbm, acc_sh.at[pl.ds(s * RPT, RPT)])
    plsc.subcore_barrier()
    base = wid * EW

    def chunk(j, _):
        off = base + j * K
        pltpu.sync_copy(src_hbm.at[pl.ds(off, K)], src_v)
        pltpu.sync_copy(dst_hbm.at[pl.ds(off, K)], dst_v)
        pltpu.async_copy(y_hbm.at[src_v], rows_v, sem).wait()
        pltpu.sync_copy(rows_v, acc_sh.at[dst_v], add=True)
        return 0

    lax.fori_loop(0, CH, chunk, 0)
    plsc.subcore_barrier()
    pltpu.sync_copy(acc_sh.at[pl.ds(s * RPT, RPT)],
                    out_hbm.at[c, pl.ds(s * RPT, RPT)])


def _tc1_body(parts_ref, x_ref, w_ref, dinv_ref, y_ref):
    deg = jnp.sum(parts_ref[...], axis=1, keepdims=True) + 1.0
    dinv = lax.rsqrt(deg)
    dinv_ref[...] = dinv
    xw = jnp.dot(x_ref[...], w_ref[...], preferred_element_type=jnp.float32)
    y_ref[...] = xw * dinv


def _tc_mid_body(acc_ref, y_ref, dinv_ref, b_ref, w_ref, out_ref):
    dinv = dinv_ref[...]
    h = jnp.maximum(dinv * (acc_ref[0] + acc_ref[1] + y_ref[...]) + b_ref[...],
                    0.0)
    out_ref[...] = jnp.dot(h, w_ref[...],
                           preferred_element_type=jnp.float32) * dinv


def _tc_out_body(parts_ref, y3_ref, dinv_ref, b3_ref, out_ref):
    a = jnp.sum(parts_ref[...], axis=1, keepdims=True) + y3_ref[...]
    out_ref[...] = jax.nn.sigmoid(dinv_ref[...] * a + b3_ref[...])


def kernel(x, edge_index, W1, b1, W2, b2, W3, b3):
    f32 = jnp.float32
    src = jnp.concatenate([edge_index[0],
                           jnp.zeros((E_PAD - E,), jnp.int32)])
    dst = jnp.concatenate([edge_index[1],
                           jnp.full((E_PAD - E,), N, jnp.int32)])
    xp = jnp.zeros((NP, D), f32).at[:N].set(x)

    ones_t = jnp.ones((NP,), f32)
    zeros_1d = jnp.zeros((NP,), f32)
    zeros_2d = jnp.zeros((RPT, D), f32)

    deg_parts = _sc_scalar(ones_t, src, dst, zeros_1d).reshape(NW, NP)

    dinv, y1 = pl.pallas_call(
        _tc1_body,
        out_shape=[jax.ShapeDtypeStruct((NP, 1), f32),
                   jax.ShapeDtypeStruct((NP, D), f32)],
    )(deg_parts.T, xp, W1)

    acc1 = _sc_rows(y1, src, dst, zeros_2d)
    y2 = pl.pallas_call(
        _tc_mid_body,
        out_shape=jax.ShapeDtypeStruct((NP, D), f32),
    )(acc1, y1, dinv, b1.reshape(1, D), W2)

    acc2 = _sc_rows(y2, src, dst, zeros_2d)
    y3 = pl.pallas_call(
        _tc_mid_body,
        out_shape=jax.ShapeDtypeStruct((NP, 1), f32),
    )(acc2, y2, dinv, b2.reshape(1, D), W3)

    acc3_parts = _sc_scalar(y3.reshape(NP), src, dst, zeros_1d).reshape(NW, NP)
    out = pl.pallas_call(
        _tc_out_body,
        out_shape=jax.ShapeDtypeStruct((NP, 1), f32),
    )(acc3_parts.T, y3, dinv, b3.reshape(1, 1))
    return out[:N]


# speedup vs baseline: 13.0964x; 1.0828x over previous
import functools

import jax
import jax.numpy as jnp
from jax import lax
from jax.experimental import pallas as pl
from jax.experimental.pallas import tpu as pltpu
from jax.experimental.pallas import tpu_sc as plsc

N = 10000
D = 128
E = 320000

NC = 2
NS = 16
NW = NC * NS
LANES = 16

NP = 10112
RPT = NP // NS
K = 128
E_PAD = 327680
EW = E_PAD // NW
CH = EW // K
NBUF = 2
NG = CH // NBUF

_mesh = plsc.VectorSubcoreMesh(core_axis_name="c", subcore_axis_name="s")
_sc_params = pltpu.CompilerParams(needs_layout_passes=False)


@functools.partial(
    pl.kernel,
    out_type=jax.ShapeDtypeStruct((NW * NP,), jnp.float32),
    mesh=_mesh,
    scratch_types=[
        pltpu.VMEM((EW,), jnp.int32),
        pltpu.VMEM((EW,), jnp.int32),
        pltpu.VMEM((NP,), jnp.float32),
        pltpu.VMEM((NP,), jnp.float32),
    ],
    compiler_params=_sc_params,
)
def _sc_scalar(table_hbm, src_hbm, dst_hbm, zeros_hbm, out_hbm,
               src_v, dst_v, table_v, acc_v):
    c = lax.axis_index("c")
    s = lax.axis_index("s")
    wid = s * NC + c
    pltpu.sync_copy(zeros_hbm, acc_v)
    pltpu.sync_copy(table_hbm, table_v)
    base = wid * EW
    pltpu.sync_copy(src_hbm.at[pl.ds(base, EW)], src_v)
    pltpu.sync_copy(dst_hbm.at[pl.ds(base, EW)], dst_v)

    def chunk(j, _):
        for i in range(K // LANES):
            o = j * K + i * LANES
            sv = src_v[pl.ds(o, LANES)]
            dv = dst_v[pl.ds(o, LANES)]
            vals = plsc.load_gather(table_v, [sv])
            plsc.addupdate_scatter(acc_v, [dv], vals)
        return 0

    lax.fori_loop(0, CH, chunk, 0)
    pltpu.sync_copy(acc_v, out_hbm.at[pl.ds(wid * NP, NP)])


@functools.partial(
    pl.kernel,
    out_type=jax.ShapeDtypeStruct((NC, NP, D), jnp.float32),
    mesh=_mesh,
    scratch_types=[
        pltpu.VMEM((EW,), jnp.int32),
        [pltpu.VMEM((K,), jnp.int32) for _ in range(NBUF)],
        [pltpu.VMEM((K, D), jnp.float32) for _ in range(NBUF)],
        pltpu.VMEM_SHARED((NP, D), jnp.float32),
        pltpu.SemaphoreType.DMA((NBUF,)),
        pltpu.SemaphoreType.DMA((NBUF,)),
        pltpu.SemaphoreType.DMA((NBUF,)),
    ],
    compiler_params=_sc_params,
)
def _sc_rows(y_hbm, src_hbm, dst_hbm, zeros_hbm, out_hbm,
             src_v, dst_bufs, row_bufs, acc_sh, dsem, gsem, ssem):
    c = lax.axis_index("c")
    s = lax.axis_index("s")
    wid = s * NC + c
    base = wid * EW
    pltpu.sync_copy(zeros_hbm, acc_sh.at[pl.ds(s * RPT, RPT)])
    pltpu.sync_copy(src_hbm.at[pl.ds(base, EW)], src_v)
    plsc.subcore_barrier()

    def start_chunk(cix, b):
        pltpu.async_copy(dst_hbm.at[pl.ds(base + cix * K, K)],
                         dst_bufs[b], dsem.at[b])
        pltpu.async_copy(y_hbm.at[src_v.at[pl.ds(cix * K, K)]],
                         row_bufs[b], gsem.at[b])

    for b in range(NBUF):
        start_chunk(b, b)

    def group(i, _):
        for b in range(NBUF):
            cix = i * NBUF + b
            pltpu.make_async_copy(dst_hbm.at[pl.ds(base + cix * K, K)],
                                  dst_bufs[b], dsem.at[b]).wait()
            pltpu.make_async_copy(y_hbm.at[src_v.at[pl.ds(cix * K, K)]],
                                  row_bufs[b], gsem.at[b]).wait()
            pltpu.async_copy(row_bufs[b], acc_sh.at[dst_bufs[b]],
                             ssem.at[b], add=True)
        for b in range(NBUF):
            pltpu.make_async_copy(row_bufs[b], acc_sh.at[dst_bufs[b]],
                                  ssem.at[b]).wait()

            @pl.when(i < NG - 1)
            def _():
                start_chunk((i + 1) * NBUF + b, b)
        return 0

    lax.fori_loop(0, NG, group, 0)
    plsc.subcore_barrier()
    pltpu.sync_copy(acc_sh.at[pl.ds(s * RPT, RPT)],
                    out_hbm.at[c, pl.ds(s * RPT, RPT)])


def _tc1_body(parts_ref, x_ref, w_ref, dinv_ref, y_ref):
    deg = jnp.sum(parts_ref[...], axis=1, keepdims=True) + 1.0
    dinv = lax.rsqrt(deg)
    dinv_ref[...] = dinv
    xw = jnp.dot(x_ref[...], w_ref[...], preferred_element_type=jnp.float32)
    y_ref[...] = xw * dinv


def _tc_mid_body(acc_ref, y_ref, dinv_ref, b_ref, w_ref, out_ref):
    dinv = dinv_ref[...]
    h = jnp.maximum(dinv * (acc_ref[0] + acc_ref[1] + y_ref[...]) + b_ref[...],
                    0.0)
    out_ref[...] = jnp.dot(h, w_ref[...],
                           preferred_element_type=jnp.float32) * dinv


def _tc_out_body(parts_ref, y3_ref, dinv_ref, b3_ref, out_ref):
    a = jnp.sum(parts_ref[...], axis=1, keepdims=True) + y3_ref[...]
    out_ref[...] = jax.nn.sigmoid(dinv_ref[...] * a + b3_ref[...])


def kernel(x, edge_index, W1, b1, W2, b2, W3, b3):
    f32 = jnp.float32
    src = jnp.concatenate([edge_index[0],
                           jnp.zeros((E_PAD - E,), jnp.int32)])
    dst = jnp.concatenate([edge_index[1],
                           jnp.full((E_PAD - E,), N, jnp.int32)])
    xp = jnp.zeros((NP, D), f32).at[:N].set(x)

    ones_t = jnp.ones((NP,), f32)
    zeros_1d = jnp.zeros((NP,), f32)
    zeros_2d = jnp.zeros((RPT, D), f32)

    deg_parts = _sc_scalar(ones_t, src, dst, zeros_1d).reshape(NW, NP)

    dinv, y1 = pl.pallas_call(
        _tc1_body,
        out_shape=[jax.ShapeDtypeStruct((NP, 1), f32),
                   jax.ShapeDtypeStruct((NP, D), f32)],
    )(deg_parts.T, xp, W1)

    acc1 = _sc_rows(y1, src, dst, zeros_2d)
    y2 = pl.pallas_call(
        _tc_mid_body,
        out_shape=jax.ShapeDtypeStruct((NP, D), f32),
    )(acc1, y1, dinv, b1.reshape(1, D), W2)

    acc2 = _sc_rows(y2, src, dst, zeros_2d)
    y3 = pl.pallas_call(
        _tc_mid_body,
        out_shape=jax.ShapeDtypeStruct((NP, 1), f32),
    )(acc2, y2, dinv, b2.reshape(1, D), W3)

    acc3_parts = _sc_scalar(y3.reshape(NP), src, dst, zeros_1d).reshape(NW, NP)
    out = pl.pallas_call(
        _tc_out_body,
        out_shape=jax.ShapeDtypeStruct((NP, 1), f32),
    )(acc3_parts.T, y3, dinv, b3.reshape(1, 1))
    return out[:N]
